# Initial kernel scaffold; baseline (speedup 1.0000x reference)
import jax
import jax.numpy as jnp
from jax.experimental import pallas as pl


def kernel(x, char_codes, char_emb, conv_w, conv_b, word_emb):
    raise NotImplementedError("write your pallas kernel here")


# speedup vs baseline: 2.6388x; 2.6388x over previous
import functools

import jax
import jax.numpy as jnp
from jax import lax
from jax.experimental import pallas as pl
from jax.experimental.pallas import tpu as pltpu
from jax.experimental.pallas import tpu_sc as plsc

V2 = 100002
L = 16
CP = 64
WP = 64
W = 3
T = L - W + 1
B, S = 4096, 50
NTOK = B * S

NW = 32
VPAD = 102400
RW = VPAD // NW
CCH = 320
NCHUNK = RW // CCH
TPW = NTOK // NW
TCH = 128
NTCH = TPW // TCH

_mesh = plsc.VectorSubcoreMesh(core_axis_name="c", subcore_axis_name="s")
_sc_params = pltpu.CompilerParams(needs_layout_passes=False)


def _m_matmul_body(ce_ref, w2_ref, m_ref):
  m_ref[...] = jnp.dot(ce_ref[...], w2_ref[...],
                       preferred_element_type=jnp.float32)


D = CP + WP


@functools.partial(
    pl.kernel, mesh=_mesh, compiler_params=_sc_params,
    out_type=jax.ShapeDtypeStruct((VPAD * D,), jnp.float32),
    scratch_types=[
        pltpu.VMEM((100 * W * CP,), jnp.float32),
        pltpu.VMEM((CP,), jnp.float32),
        pltpu.VMEM((CCH * L,), jnp.int32),
        pltpu.VMEM((CCH * WP,), jnp.float32),
        pltpu.VMEM((CCH * D,), jnp.float32),
    ])
def _table_kernel(m_hbm, b_hbm, codes_hbm, wemb_hbm, table_hbm,
                  m_v, b_v, codes_v, wemb_v, out_v):
  wid = lax.axis_index("s") * 2 + lax.axis_index("c")
  base = wid * RW
  pltpu.sync_copy(m_hbm, m_v)
  pltpu.sync_copy(b_hbm, b_v)
  lanes = lax.iota(jnp.int32, 16)

  def chunk_body(ci, carry):
    row0 = base + ci * CCH
    pltpu.sync_copy(codes_hbm.at[pl.ds(row0 * L, CCH * L)], codes_v)
    pltpu.sync_copy(wemb_hbm.at[pl.ds(row0 * WP, CCH * WP)], wemb_v)

    def group_body(g, carry):
      riota = lanes + g * 16
      ms = []
      for j in range(L):
        cvec = plsc.load_gather(codes_v, [riota * L + j])
        ms.append(cvec * (W * CP))

      def o_body(o, carry):
        acc = jnp.full((16,), -3.4e38, jnp.float32)
        for t in range(T):
          s = plsc.load_gather(m_v, [ms[t] + o])
          s = s + plsc.load_gather(m_v, [ms[t + 1] + (CP + o)])
          s = s + plsc.load_gather(m_v, [ms[t + 2] + (2 * CP + o)])
          acc = jnp.maximum(acc, s)
        bo = plsc.load_gather(b_v, [lanes * 0 + o])
        res = jnp.maximum(acc + bo, 0.0)
        plsc.store_scatter(out_v, [riota * D + o], res)
        return carry

      return lax.fori_loop(0, CP, o_body, carry)

    carry = lax.fori_loop(0, CCH // 16, group_body, carry)

    def w_body(r, carry):
      for q in range(WP // 16):
        out_v[pl.ds(r * D + CP + q * 16, 16)] = wemb_v[pl.ds(r * WP + q * 16, 16)]
      return carry

    carry = lax.fori_loop(0, CCH, w_body, carry)
    pltpu.sync_copy(out_v, table_hbm.at[pl.ds(row0 * D, CCH * D)])
    return carry

  lax.fori_loop(0, NCHUNK, chunk_body, 0)


@functools.partial(
    pl.kernel, mesh=_mesh, compiler_params=_sc_params,
    out_type=jax.ShapeDtypeStruct((NTOK, D), jnp.float32),
    scratch_types=[
        pltpu.VMEM((TCH,), jnp.int32),
        pltpu.VMEM((TCH, D), jnp.float32),
        pltpu.SemaphoreType.DMA,
    ])
def _gather_kernel(x_hbm, tab_hbm, out_hbm, idx_v, row_v, sem):
  wid = lax.axis_index("s") * 2 + lax.axis_index("c")
  base = wid * TPW

  def it_body(i, carry):
    t0 = base + i * TCH
    pltpu.sync_copy(x_hbm.at[pl.ds(t0, TCH)], idx_v)
    pltpu.async_copy(tab_hbm.at[idx_v], row_v, sem).wait()
    pltpu.sync_copy(row_v, out_hbm.at[pl.ds(t0, TCH), :])
    return carry

  lax.fori_loop(0, NTCH, it_body, 0)


def kernel(x, char_codes, char_emb, conv_w, conv_b, word_emb):
  ce_p = jnp.pad(char_emb.astype(jnp.float32), ((0, 28), (0, 0)))
  w2 = conv_w.astype(jnp.float32).transpose(1, 2, 0).reshape(16, W * CP)
  w2_p = jnp.pad(w2, ((0, 0), (0, 64)))
  m_pad = pl.pallas_call(
      _m_matmul_body,
      out_shape=jax.ShapeDtypeStruct((128, 256), jnp.float32),
  )(ce_p, w2_p)
  m_flat = m_pad[:100, :W * CP].reshape(100 * W * CP)

  codes_p = jnp.pad(char_codes.astype(jnp.int32),
                    ((0, VPAD - V2), (0, 0))).reshape(VPAD * L)
  wemb_p = jnp.pad(word_emb.astype(jnp.float32),
                   ((0, VPAD - V2), (0, 0))).reshape(VPAD * WP)
  tab = _table_kernel(m_flat, conv_b.astype(jnp.float32),
                      codes_p, wemb_p).reshape(VPAD, D)

  xf = jnp.maximum(x.reshape(-1).astype(jnp.int32), 0)
  out = _gather_kernel(xf, tab)
  return out.reshape(B, S, D)


# speedup vs baseline: 14.2903x; 5.4154x over previous
import functools

import jax
import jax.numpy as jnp
from jax import lax
from jax.experimental import pallas as pl
from jax.experimental.pallas import tpu as pltpu
from jax.experimental.pallas import tpu_sc as plsc

V2 = 100002
L = 16
CP = 64
WP = 64
W = 3
T = L - W + 1
B, S = 4096, 50
NTOK = B * S

NW = 32
VPAD = 102400
RW = VPAD // NW
CCH = 320
NCHUNK = RW // CCH
TPW = NTOK // NW
TCH = 128
NTCH = TPW // TCH

_mesh = plsc.VectorSubcoreMesh(core_axis_name="c", subcore_axis_name="s")
_sc_params = pltpu.CompilerParams(needs_layout_passes=False)


def _m_matmul_body(ce_ref, w2_ref, m_ref):
  m_ref[...] = jnp.dot(ce_ref[...], w2_ref[...],
                       preferred_element_type=jnp.float32)


D = CP + WP


@functools.partial(
    pl.kernel, mesh=_mesh, compiler_params=_sc_params,
    out_type=jax.ShapeDtypeStruct((VPAD * D,), jnp.float32),
    scratch_types=[
        pltpu.VMEM((100 * W * CP,), jnp.float32),
        pltpu.VMEM((CP,), jnp.float32),
        pltpu.VMEM((CCH * L,), jnp.int32),
        pltpu.VMEM((CCH * WP,), jnp.float32),
        pltpu.VMEM((CCH * D,), jnp.float32),
    ])
def _table_kernel(m_hbm, b_hbm, codes_hbm, wemb_hbm, table_hbm,
                  m_v, b_v, codes_v, wemb_v, out_v):
  wid = lax.axis_index("s") * 2 + lax.axis_index("c")
  base = wid * RW
  pltpu.sync_copy(m_hbm, m_v)
  pltpu.sync_copy(b_hbm, b_v)
  bias = [b_v[pl.ds(q * 16, 16)] for q in range(CP // 16)]

  def chunk_body(ci, carry):
    row0 = base + ci * CCH
    pltpu.sync_copy(codes_hbm.at[pl.ds(row0 * L, CCH * L)], codes_v)
    pltpu.sync_copy(wemb_hbm.at[pl.ds(row0 * WP, CCH * WP)], wemb_v)

    def row_body(r, carry):
      cv = codes_v[pl.ds(r * L, L)]
      cs = [cv[j] * (W * CP) for j in range(L)]
      for q in range(CP // 16):
        acc = jnp.full((16,), -3.4e38, jnp.float32)
        for t in range(T):
          v = m_v[pl.ds(cs[t] + q * 16, 16)]
          v = v + m_v[pl.ds(cs[t + 1] + (CP + q * 16), 16)]
          v = v + m_v[pl.ds(cs[t + 2] + (2 * CP + q * 16), 16)]
          acc = jnp.maximum(acc, v)
        res = jnp.maximum(acc + bias[q], 0.0)
        out_v[pl.ds(r * D + q * 16, 16)] = res
        out_v[pl.ds(r * D + CP + q * 16, 16)] = wemb_v[pl.ds(r * WP + q * 16, 16)]
      return carry

    carry = lax.fori_loop(0, CCH, row_body, carry)
    pltpu.sync_copy(out_v, table_hbm.at[pl.ds(row0 * D, CCH * D)])
    return carry

  lax.fori_loop(0, NCHUNK, chunk_body, 0)


@functools.partial(
    pl.kernel, mesh=_mesh, compiler_params=_sc_params,
    out_type=jax.ShapeDtypeStruct((NTOK, D), jnp.float32),
    scratch_types=[
        pltpu.VMEM((TCH,), jnp.int32),
        pltpu.VMEM((TCH, D), jnp.float32),
        pltpu.SemaphoreType.DMA,
    ])
def _gather_kernel(x_hbm, tab_hbm, out_hbm, idx_v, row_v, sem):
  wid = lax.axis_index("s") * 2 + lax.axis_index("c")
  base = wid * TPW

  def it_body(i, carry):
    t0 = base + i * TCH
    pltpu.sync_copy(x_hbm.at[pl.ds(t0, TCH)], idx_v)
    pltpu.async_copy(tab_hbm.at[idx_v], row_v, sem).wait()
    pltpu.sync_copy(row_v, out_hbm.at[pl.ds(t0, TCH), :])
    return carry

  lax.fori_loop(0, NTCH, it_body, 0)


def kernel(x, char_codes, char_emb, conv_w, conv_b, word_emb):
  ce_p = jnp.pad(char_emb.astype(jnp.float32), ((0, 28), (0, 0)))
  w2 = conv_w.astype(jnp.float32).transpose(1, 2, 0).reshape(16, W * CP)
  w2_p = jnp.pad(w2, ((0, 0), (0, 64)))
  m_pad = pl.pallas_call(
      _m_matmul_body,
      out_shape=jax.ShapeDtypeStruct((128, 256), jnp.float32),
  )(ce_p, w2_p)
  m_flat = m_pad[:100, :W * CP].reshape(100 * W * CP)

  codes_p = jnp.pad(char_codes.astype(jnp.int32),
                    ((0, VPAD - V2), (0, 0))).reshape(VPAD * L)
  wemb_p = jnp.pad(word_emb.astype(jnp.float32),
                   ((0, VPAD - V2), (0, 0))).reshape(VPAD * WP)
  tab = _table_kernel(m_flat, conv_b.astype(jnp.float32),
                      codes_p, wemb_p).reshape(VPAD, D)

  xf = jnp.maximum(x.reshape(-1).astype(jnp.int32), 0)
  out = _gather_kernel(xf, tab)
  return out.reshape(B, S, D)


# speedup vs baseline: 16.2264x; 1.1355x over previous
import functools

import jax
import jax.numpy as jnp
from jax import lax
from jax.experimental import pallas as pl
from jax.experimental.pallas import tpu as pltpu
from jax.experimental.pallas import tpu_sc as plsc

V2 = 100002
L = 16
CP = 64
WP = 64
W = 3
T = L - W + 1
B, S = 4096, 50
NTOK = B * S

NW = 32
RW = 3200
CCH = 320
NCHUNK = RW // CCH
TPW = NTOK // NW
TCH = 128
NTCH = TPW // TCH

_mesh = plsc.VectorSubcoreMesh(core_axis_name="c", subcore_axis_name="s")
_sc_params = pltpu.CompilerParams(needs_layout_passes=False)


def _m_matmul_body(ce_ref, w2_ref, m_ref):
  m_ref[...] = jnp.dot(ce_ref[...], w2_ref[...],
                       preferred_element_type=jnp.float32)


D = CP + WP


@functools.partial(
    pl.kernel, mesh=_mesh, compiler_params=_sc_params,
    out_type=jax.ShapeDtypeStruct((V2 * D,), jnp.float32),
    scratch_types=[
        pltpu.VMEM((100 * W * CP,), jnp.float32),
        pltpu.VMEM((CP,), jnp.float32),
        pltpu.VMEM((CCH * L,), jnp.int32),
        pltpu.VMEM((CCH * WP,), jnp.float32),
        pltpu.VMEM((CCH * D,), jnp.float32),
    ])
def _table_kernel(m_hbm, b_hbm, codes_hbm, wemb_hbm, table_hbm,
                  m_v, b_v, codes_v, wemb_v, out_v):
  wid = lax.axis_index("s") * 2 + lax.axis_index("c")
  base = wid * RW
  pltpu.sync_copy(m_hbm, m_v)
  pltpu.sync_copy(b_hbm, b_v)
  bias = [b_v[pl.ds(q * 16, 16)] for q in range(CP // 16)]

  def chunk_body(ci, carry):
    row0 = jnp.minimum(base + ci * CCH, V2 - CCH)
    pltpu.sync_copy(codes_hbm.at[pl.ds(row0 * L, CCH * L)], codes_v)
    pltpu.sync_copy(wemb_hbm.at[pl.ds(row0 * WP, CCH * WP)], wemb_v)

    def row_body(r, carry):
      cv = codes_v[pl.ds(r * L, L)]
      cs = [cv[j] * (W * CP) for j in range(L)]
      for q in range(CP // 16):
        acc = jnp.full((16,), -3.4e38, jnp.float32)
        for t in range(T):
          v = m_v[pl.ds(cs[t] + q * 16, 16)]
          v = v + m_v[pl.ds(cs[t + 1] + (CP + q * 16), 16)]
          v = v + m_v[pl.ds(cs[t + 2] + (2 * CP + q * 16), 16)]
          acc = jnp.maximum(acc, v)
        res = jnp.maximum(acc + bias[q], 0.0)
        out_v[pl.ds(r * D + q * 16, 16)] = res
        out_v[pl.ds(r * D + CP + q * 16, 16)] = wemb_v[pl.ds(r * WP + q * 16, 16)]
      return carry

    carry = lax.fori_loop(0, CCH, row_body, carry)
    pltpu.sync_copy(out_v, table_hbm.at[pl.ds(row0 * D, CCH * D)])
    return carry

  lax.fori_loop(0, NCHUNK, chunk_body, 0)


@functools.partial(
    pl.kernel, mesh=_mesh, compiler_params=_sc_params,
    out_type=jax.ShapeDtypeStruct((NTOK, D), jnp.float32),
    scratch_types=[
        pltpu.VMEM((TPW,), jnp.int32),
        pltpu.VMEM((TCH, D), jnp.float32),
        pltpu.VMEM((TCH, D), jnp.float32),
        pltpu.SemaphoreType.DMA,
        pltpu.SemaphoreType.DMA,
        pltpu.SemaphoreType.DMA,
        pltpu.SemaphoreType.DMA,
    ])
def _gather_kernel(x_hbm, tab_hbm, out_hbm, idx_v, buf0, buf1,
                   g_sem0, g_sem1, s_sem0, s_sem1):
  wid = lax.axis_index("s") * 2 + lax.axis_index("c")
  base = wid * TPW
  pltpu.sync_copy(x_hbm.at[pl.ds(base, TPW)], idx_v)
  bufs = (buf0, buf1)
  g_sems = (g_sem0, g_sem1)
  s_sems = (s_sem0, s_sem1)

  def gather(i):
    b = i % 2
    return pltpu.async_copy(
        tab_hbm.at[idx_v.at[pl.ds(i * TCH, TCH)]], bufs[b], g_sems[b])

  def scatter(i):
    b = i % 2
    return pltpu.async_copy(
        bufs[b], out_hbm.at[pl.ds(base + i * TCH, TCH), :], s_sems[b])

  gathers = [None] * NTCH
  scatters = [None] * NTCH
  gathers[0] = gather(0)
  for i in range(1, NTCH):
    if i >= 2:
      scatters[i - 2].wait()
    gathers[i] = gather(i)
    gathers[i - 1].wait()
    scatters[i - 1] = scatter(i - 1)
  gathers[NTCH - 1].wait()
  scatters[NTCH - 1] = scatter(NTCH - 1)
  scatters[NTCH - 2].wait()
  scatters[NTCH - 1].wait()


def kernel(x, char_codes, char_emb, conv_w, conv_b, word_emb):
  ce_p = jnp.pad(char_emb.astype(jnp.float32), ((0, 28), (0, 0)))
  w2 = conv_w.astype(jnp.float32).transpose(1, 2, 0).reshape(16, W * CP)
  w2_p = jnp.pad(w2, ((0, 0), (0, 64)))
  m_pad = pl.pallas_call(
      _m_matmul_body,
      out_shape=jax.ShapeDtypeStruct((128, 256), jnp.float32),
  )(ce_p, w2_p)
  m_flat = m_pad[:100, :W * CP].reshape(100 * W * CP)

  codes_f = char_codes.astype(jnp.int32).reshape(V2 * L)
  wemb_f = word_emb.astype(jnp.float32).reshape(V2 * WP)
  tab = _table_kernel(m_flat, conv_b.astype(jnp.float32),
                      codes_f, wemb_f).reshape(V2, D)

  xf = jnp.maximum(x.reshape(-1).astype(jnp.int32), 0)
  out = _gather_kernel(xf, tab)
  return out.reshape(B, S, D)


# speedup vs baseline: 16.4178x; 1.0118x over previous
import functools

import jax
import jax.numpy as jnp
from jax import lax
from jax.experimental import pallas as pl
from jax.experimental.pallas import tpu as pltpu
from jax.experimental.pallas import tpu_sc as plsc

V2 = 100002
L = 16
CP = 64
WP = 64
W = 3
T = L - W + 1
B, S = 4096, 50
NTOK = B * S

NW = 32
RW = 3200
CCH = 320
NCHUNK = RW // CCH
TPW = NTOK // NW
TCH = 128
NTCH = TPW // TCH

_mesh = plsc.VectorSubcoreMesh(core_axis_name="c", subcore_axis_name="s")
_sc_params = pltpu.CompilerParams(needs_layout_passes=False)


def _m_matmul_body(ce_ref, w2_ref, m_ref):
  m_ref[...] = jnp.dot(ce_ref[...], w2_ref[...],
                       preferred_element_type=jnp.float32)


D = CP + WP


@functools.partial(
    pl.kernel, mesh=_mesh, compiler_params=_sc_params,
    out_type=jax.ShapeDtypeStruct((V2 * D,), jnp.float32),
    scratch_types=[
        pltpu.VMEM((100 * W * CP,), jnp.float32),
        pltpu.VMEM((CP,), jnp.float32),
        pltpu.VMEM((CCH * L,), jnp.int32),
        pltpu.VMEM((CCH * WP,), jnp.float32),
        pltpu.VMEM((CCH * D,), jnp.float32),
        pltpu.VMEM((CCH * D,), jnp.float32),
        pltpu.SemaphoreType.DMA,
        pltpu.SemaphoreType.DMA,
    ])
def _table_kernel(m_hbm, b_hbm, codes_hbm, wemb_hbm, table_hbm,
                  m_v, b_v, codes_v, wemb_v, out_v0, out_v1, osem0, osem1):
  wid = lax.axis_index("s") * 2 + lax.axis_index("c")
  base = wid * RW
  pltpu.sync_copy(m_hbm, m_v)
  pltpu.sync_copy(b_hbm, b_v)
  bias = [b_v[pl.ds(q * 16, 16)] for q in range(CP // 16)]
  out_bufs = (out_v0, out_v1)
  osems = (osem0, osem1)

  writes = [None] * NCHUNK
  for ci in range(NCHUNK):
    out_v = out_bufs[ci % 2]
    if ci >= 2:
      writes[ci - 2].wait()
    row0 = jnp.minimum(base + ci * CCH, V2 - CCH)
    pltpu.sync_copy(codes_hbm.at[pl.ds(row0 * L, CCH * L)], codes_v)
    pltpu.sync_copy(wemb_hbm.at[pl.ds(row0 * WP, CCH * WP)], wemb_v)

    def row_body(r, carry, out_v=out_v):
      cv = codes_v[pl.ds(r * L, L)]
      cs = [cv[j] * (W * CP) for j in range(L)]
      for q in range(CP // 16):
        acc = jnp.full((16,), -3.4e38, jnp.float32)
        for t in range(T):
          v = m_v[pl.ds(cs[t] + q * 16, 16)]
          v = v + m_v[pl.ds(cs[t + 1] + (CP + q * 16), 16)]
          v = v + m_v[pl.ds(cs[t + 2] + (2 * CP + q * 16), 16)]
          acc = jnp.maximum(acc, v)
        res = jnp.maximum(acc + bias[q], 0.0)
        out_v[pl.ds(r * D + q * 16, 16)] = res
        out_v[pl.ds(r * D + CP + q * 16, 16)] = wemb_v[pl.ds(r * WP + q * 16, 16)]
      return carry

    lax.fori_loop(0, CCH, row_body, 0)
    writes[ci] = pltpu.async_copy(
        out_v, table_hbm.at[pl.ds(row0 * D, CCH * D)], osems[ci % 2])
  writes[NCHUNK - 2].wait()
  writes[NCHUNK - 1].wait()


@functools.partial(
    pl.kernel, mesh=_mesh, compiler_params=_sc_params,
    out_type=jax.ShapeDtypeStruct((NTOK, D), jnp.float32),
    scratch_types=[
        pltpu.VMEM((TPW,), jnp.int32),
        pltpu.VMEM((TCH, D), jnp.float32),
        pltpu.VMEM((TCH, D), jnp.float32),
        pltpu.SemaphoreType.DMA,
        pltpu.SemaphoreType.DMA,
        pltpu.SemaphoreType.DMA,
        pltpu.SemaphoreType.DMA,
    ])
def _gather_kernel(x_hbm, tab_hbm, out_hbm, idx_v, buf0, buf1,
                   g_sem0, g_sem1, s_sem0, s_sem1):
  wid = lax.axis_index("s") * 2 + lax.axis_index("c")
  base = wid * TPW
  pltpu.sync_copy(x_hbm.at[pl.ds(base, TPW)], idx_v)
  bufs = (buf0, buf1)
  g_sems = (g_sem0, g_sem1)
  s_sems = (s_sem0, s_sem1)

  def gather(i):
    b = i % 2
    return pltpu.async_copy(
        tab_hbm.at[idx_v.at[pl.ds(i * TCH, TCH)]], bufs[b], g_sems[b])

  def scatter(i):
    b = i % 2
    return pltpu.async_copy(
        bufs[b], out_hbm.at[pl.ds(base + i * TCH, TCH), :], s_sems[b])

  gathers = [None] * NTCH
  scatters = [None] * NTCH
  gathers[0] = gather(0)
  for i in range(1, NTCH):
    if i >= 2:
      scatters[i - 2].wait()
    gathers[i] = gather(i)
    gathers[i - 1].wait()
    scatters[i - 1] = scatter(i - 1)
  gathers[NTCH - 1].wait()
  scatters[NTCH - 1] = scatter(NTCH - 1)
  scatters[NTCH - 2].wait()
  scatters[NTCH - 1].wait()


def kernel(x, char_codes, char_emb, conv_w, conv_b, word_emb):
  ce_p = jnp.pad(char_emb.astype(jnp.float32), ((0, 28), (0, 0)))
  w2 = conv_w.astype(jnp.float32).transpose(1, 2, 0).reshape(16, W * CP)
  w2_p = jnp.pad(w2, ((0, 0), (0, 64)))
  m_pad = pl.pallas_call(
      _m_matmul_body,
      out_shape=jax.ShapeDtypeStruct((128, 256), jnp.float32),
  )(ce_p, w2_p)
  m_flat = m_pad[:100, :W * CP].reshape(100 * W * CP)

  codes_f = char_codes.astype(jnp.int32).reshape(V2 * L)
  wemb_f = word_emb.astype(jnp.float32).reshape(V2 * WP)
  tab = _table_kernel(m_flat, conv_b.astype(jnp.float32),
                      codes_f, wemb_f).reshape(V2, D)

  xf = jnp.maximum(x.reshape(-1).astype(jnp.int32), 0)
  out = _gather_kernel(xf, tab)
  return out.reshape(B, S, D)
